# initial kernel scaffold (unmeasured)
import jax
import jax.numpy as jnp
from jax import lax
from jax.experimental import pallas as pl
from jax.experimental.pallas import tpu as pltpu

N_DEV = 4
BLK = 64


def kernel(x, Wq, K_ext, V_ext, Wo):
    B, Sq, D = x.shape
    _, Skv, Hq, Dh = K_ext.shape
    HD = Hq * Dh

    def body(x_ref, wq_ref, k_ref, v_ref, wo_ref, out_ref,
             kvbuf, send_sems, recv_sems):
        my = lax.axis_index("i")
        left = lax.rem(my - 1 + N_DEV, N_DEV)
        right = lax.rem(my + 1, N_DEV)

        barrier_sem = pltpu.get_barrier_semaphore()
        for nbr in (left, right):
            pl.semaphore_signal(
                barrier_sem, inc=1,
                device_id=(nbr,), device_id_type=pl.DeviceIdType.MESH,
            )
        pl.semaphore_wait(barrier_sem, 2)

        kvbuf[0, 0] = k_ref[:]
        kvbuf[0, 1] = v_ref[:]

        q = jnp.dot(
            x_ref[:].reshape(B * Sq, D), wq_ref[:],
            preferred_element_type=jnp.float32,
        ) * 0.125
        q = q.reshape(B, Sq, Hq, Dh)

        m = jnp.full((B, Hq, Sq), -1e30, jnp.float32)
        l = jnp.zeros((B, Hq, Sq), jnp.float32)
        acc = jnp.zeros((B, Hq, Sq, Dh), jnp.float32)

        iq = lax.broadcasted_iota(jnp.int32, (Sq, Skv), 0)
        ik = lax.broadcasted_iota(jnp.int32, (Sq, Skv), 1)
        qb = my * (Sq // BLK) + iq // BLK

        def process(slot, origin, m, l, acc):
            kb = origin * (Skv // BLK) + ik // BLK
            mask = (qb == kb) | (kb == 0) | (lax.rem(qb + kb, 3) == 0)
            new_m, new_l, new_acc = [], [], []
            for b in range(B):
                k_b = kvbuf[slot, 0, b]
                v_b = kvbuf[slot, 1, b]
                s = lax.dot_general(
                    q[b], k_b, (((2,), (2,)), ((1,), (1,))),
                    preferred_element_type=jnp.float32,
                )
                m_c = jnp.where(mask[None], s, -1e30).max(axis=-1)
                m_n = jnp.maximum(m[b], m_c)
                alpha = jnp.exp(m[b] - m_n)
                p = jnp.where(mask[None], jnp.exp(s - m_n[..., None]), 0.0)
                pv = lax.dot_general(
                    p, v_b, (((2,), (0,)), ((0,), (1,))),
                    preferred_element_type=jnp.float32,
                )
                new_m.append(m_n)
                new_l.append(l[b] * alpha + p.sum(axis=-1))
                new_acc.append(acc[b] * alpha[..., None] + pv)
            return (jnp.stack(new_m), jnp.stack(new_l), jnp.stack(new_acc))

        for s in range(N_DEV - 1):
            rdma = pltpu.make_async_remote_copy(
                src_ref=kvbuf.at[s],
                dst_ref=kvbuf.at[s + 1],
                send_sem=send_sems.at[s],
                recv_sem=recv_sems.at[s + 1],
                device_id=(right,),
                device_id_type=pl.DeviceIdType.MESH,
            )
            rdma.start()
            m, l, acc = process(s, lax.rem(my - s + N_DEV, N_DEV), m, l, acc)
            rdma.wait()
        m, l, acc = process(
            N_DEV - 1, lax.rem(my - (N_DEV - 1) + N_DEV, N_DEV), m, l, acc
        )

        for b in range(B):
            ctx = (acc[b] / l[b][..., None]).transpose(1, 0, 2)
            out_ref[b] = jnp.dot(
                ctx.reshape(Sq, HD), wo_ref[:],
                preferred_element_type=jnp.float32,
            )

    return pl.pallas_call(
        body,
        out_shape=jax.ShapeDtypeStruct((B, Sq, D), jnp.float32),
        in_specs=[pl.BlockSpec(memory_space=pltpu.VMEM)] * 5,
        out_specs=pl.BlockSpec(memory_space=pltpu.VMEM),
        scratch_shapes=[
            pltpu.VMEM((N_DEV, 2, B, Skv, Hq, Dh), jnp.float32),
            pltpu.SemaphoreType.DMA((N_DEV,)),
            pltpu.SemaphoreType.DMA((N_DEV,)),
        ],
        compiler_params=pltpu.CompilerParams(collective_id=0),
    )(x, Wq, K_ext, V_ext, Wo)


# baseline (device time: 175254 ns/iter reference)
import jax
import jax.numpy as jnp
from jax import lax
from jax.experimental import pallas as pl
from jax.experimental.pallas import tpu as pltpu

N_DEV = 4
BLK = 64


def kernel(x, Wq, K_ext, V_ext, Wo):
    B, Sq, D = x.shape
    _, Skv, Hq, Dh = K_ext.shape
    HD = Hq * Dh

    def body(x_ref, wq_ref, k_ref, v_ref, wo_ref, out_ref,
             kvbuf, q_scr, acc_scr, m_scr, l_scr, send_sems, recv_sems):
        my = lax.axis_index("i")
        left = lax.rem(my - 1 + N_DEV, N_DEV)
        right = lax.rem(my + 1, N_DEV)

        barrier_sem = pltpu.get_barrier_semaphore()
        for nbr in (left, right):
            pl.semaphore_signal(
                barrier_sem, inc=1,
                device_id=(nbr,), device_id_type=pl.DeviceIdType.MESH,
            )
        pl.semaphore_wait(barrier_sem, 2)

        kvbuf[0, 0] = k_ref[:].reshape(B, Skv, HD)
        kvbuf[0, 1] = v_ref[:].reshape(B, Skv, HD)

        q_scr[:] = (jnp.dot(
            x_ref[:].reshape(B * Sq, D), wq_ref[:],
            preferred_element_type=jnp.float32,
        ) * 0.125).reshape(B, Sq, HD)

        m_scr[:] = jnp.full((B, Hq, Sq), -1e30, jnp.float32)
        l_scr[:] = jnp.zeros((B, Hq, Sq), jnp.float32)
        acc_scr[:] = jnp.zeros((B, Sq, HD), jnp.float32)

        iq = lax.broadcasted_iota(jnp.int32, (Sq, Skv), 0)
        ik = lax.broadcasted_iota(jnp.int32, (Sq, Skv), 1)
        qb = my * (Sq // BLK) + iq // BLK

        def process(slot, origin):
            kb = origin * (Skv // BLK) + ik // BLK
            mask = (qb == kb) | (kb == 0) | (lax.rem(qb + kb, 3) == 0)
            for b in range(B):
                k_b = kvbuf[slot, 0, b]
                v_b = kvbuf[slot, 1, b]
                for h in range(Hq):
                    sl = slice(h * Dh, (h + 1) * Dh)
                    s = lax.dot_general(
                        q_scr[b, :, sl], k_b[:, sl],
                        (((1,), (1,)), ((), ())),
                        preferred_element_type=jnp.float32,
                    )
                    m_prev = m_scr[b, h]
                    m_c = jnp.where(mask, s, -1e30).max(axis=-1)
                    m_n = jnp.maximum(m_prev, m_c)
                    alpha = jnp.exp(m_prev - m_n)
                    p = jnp.where(mask, jnp.exp(s - m_n[:, None]), 0.0)
                    pv = lax.dot_general(
                        p, v_b[:, sl], (((1,), (0,)), ((), ())),
                        preferred_element_type=jnp.float32,
                    )
                    m_scr[b, h] = m_n
                    l_scr[b, h] = l_scr[b, h] * alpha + p.sum(axis=-1)
                    acc_scr[b, :, sl] = acc_scr[b, :, sl] * alpha[:, None] + pv

        for s in range(N_DEV - 1):
            rdma = pltpu.make_async_remote_copy(
                src_ref=kvbuf.at[s],
                dst_ref=kvbuf.at[s + 1],
                send_sem=send_sems.at[s],
                recv_sem=recv_sems.at[s + 1],
                device_id=(right,),
                device_id_type=pl.DeviceIdType.MESH,
            )
            rdma.start()
            process(s, lax.rem(my - s + N_DEV, N_DEV))
            rdma.wait()
        process(N_DEV - 1, lax.rem(my - (N_DEV - 1) + N_DEV, N_DEV))

        for b in range(B):
            for h in range(Hq):
                sl = slice(h * Dh, (h + 1) * Dh)
                acc_scr[b, :, sl] = acc_scr[b, :, sl] / l_scr[b, h][:, None]
            out_ref[b] = jnp.dot(
                acc_scr[b], wo_ref[:], preferred_element_type=jnp.float32,
            )

    return pl.pallas_call(
        body,
        out_shape=jax.ShapeDtypeStruct((B, Sq, D), jnp.float32),
        in_specs=[pl.BlockSpec(memory_space=pltpu.VMEM)] * 5,
        out_specs=pl.BlockSpec(memory_space=pltpu.VMEM),
        scratch_shapes=[
            pltpu.VMEM((N_DEV, 2, B, Skv, HD), jnp.float32),
            pltpu.VMEM((B, Sq, HD), jnp.float32),
            pltpu.VMEM((B, Sq, HD), jnp.float32),
            pltpu.VMEM((B, Hq, Sq), jnp.float32),
            pltpu.VMEM((B, Hq, Sq), jnp.float32),
            pltpu.SemaphoreType.DMA((N_DEV,)),
            pltpu.SemaphoreType.DMA((N_DEV,)),
        ],
        compiler_params=pltpu.CompilerParams(collective_id=0),
    )(x, Wq, K_ext, V_ext, Wo)


# device time: 107448 ns/iter; 1.6311x vs baseline; 1.6311x over previous
import jax
import jax.numpy as jnp
from jax import lax
from jax.experimental import pallas as pl
from jax.experimental.pallas import tpu as pltpu

N_DEV = 4
BLK = 64


def kernel(x, Wq, K_ext, V_ext, Wo):
    B, Sq, D = x.shape
    _, Skv, Hq, Dh = K_ext.shape
    HD = Hq * Dh

    def body(x_ref, wq_ref, k_ref, v_ref, wo_ref, out_ref,
             rbuf, lbuf, q_scr, acc_scr, m_scr, l_scr,
             r_send, r_recv, l_send, l_recv):
        my = lax.axis_index("i")
        left = lax.rem(my - 1 + N_DEV, N_DEV)
        right = lax.rem(my + 1, N_DEV)

        barrier_sem = pltpu.get_barrier_semaphore()
        for nbr in (left, right):
            pl.semaphore_signal(
                barrier_sem, inc=1,
                device_id=(nbr,), device_id_type=pl.DeviceIdType.MESH,
            )
        pl.semaphore_wait(barrier_sem, 2)

        rbuf[0, 0] = k_ref[0].reshape(Skv, HD)
        rbuf[0, 1] = v_ref[0].reshape(Skv, HD)
        lbuf[0, 0] = k_ref[1].reshape(Skv, HD)
        lbuf[0, 1] = v_ref[1].reshape(Skv, HD)

        q_scr[:] = (jnp.dot(
            x_ref[:].reshape(B * Sq, D), wq_ref[:],
            preferred_element_type=jnp.float32,
        ) * 0.125).reshape(B, Sq, HD)

        m_scr[:] = jnp.full((B, Hq, Sq), -1e30, jnp.float32)
        l_scr[:] = jnp.zeros((B, Hq, Sq), jnp.float32)
        acc_scr[:] = jnp.zeros((B, Sq, HD), jnp.float32)

        iq = lax.broadcasted_iota(jnp.int32, (Sq, Skv), 0)
        ik = lax.broadcasted_iota(jnp.int32, (Sq, Skv), 1)
        qb = my * (Sq // BLK) + iq // BLK

        def process(buf, slot, b, origin):
            kb = origin * (Skv // BLK) + ik // BLK
            mask = (qb == kb) | (kb == 0) | (lax.rem(qb + kb, 3) == 0)
            k_b = buf[slot, 0]
            v_b = buf[slot, 1]
            for h in range(Hq):
                sl = slice(h * Dh, (h + 1) * Dh)
                s = lax.dot_general(
                    q_scr[b, :, sl], k_b[:, sl],
                    (((1,), (1,)), ((), ())),
                    preferred_element_type=jnp.float32,
                )
                m_prev = m_scr[b, h]
                m_c = jnp.where(mask, s, -1e30).max(axis=-1)
                m_n = jnp.maximum(m_prev, m_c)
                alpha = jnp.exp(m_prev - m_n)
                p = jnp.where(mask, jnp.exp(s - m_n[:, None]), 0.0)
                pv = lax.dot_general(
                    p, v_b[:, sl], (((1,), (0,)), ((), ())),
                    preferred_element_type=jnp.float32,
                )
                m_scr[b, h] = m_n
                l_scr[b, h] = l_scr[b, h] * alpha + p.sum(axis=-1)
                acc_scr[b, :, sl] = acc_scr[b, :, sl] * alpha[:, None] + pv

        for s in range(N_DEV - 1):
            r_rdma = pltpu.make_async_remote_copy(
                src_ref=rbuf.at[s], dst_ref=rbuf.at[s + 1],
                send_sem=r_send.at[s], recv_sem=r_recv.at[s + 1],
                device_id=(right,), device_id_type=pl.DeviceIdType.MESH,
            )
            l_rdma = pltpu.make_async_remote_copy(
                src_ref=lbuf.at[s], dst_ref=lbuf.at[s + 1],
                send_sem=l_send.at[s], recv_sem=l_recv.at[s + 1],
                device_id=(left,), device_id_type=pl.DeviceIdType.MESH,
            )
            r_rdma.start()
            l_rdma.start()
            process(rbuf, s, 0, lax.rem(my - s + N_DEV, N_DEV))
            process(lbuf, s, 1, lax.rem(my + s, N_DEV))
            r_rdma.wait()
            l_rdma.wait()
        s = N_DEV - 1
        process(rbuf, s, 0, lax.rem(my - s + N_DEV, N_DEV))
        process(lbuf, s, 1, lax.rem(my + s, N_DEV))

        for b in range(B):
            for h in range(Hq):
                sl = slice(h * Dh, (h + 1) * Dh)
                acc_scr[b, :, sl] = acc_scr[b, :, sl] / l_scr[b, h][:, None]
            out_ref[b] = jnp.dot(
                acc_scr[b], wo_ref[:], preferred_element_type=jnp.float32,
            )

    return pl.pallas_call(
        body,
        out_shape=jax.ShapeDtypeStruct((B, Sq, D), jnp.float32),
        in_specs=[pl.BlockSpec(memory_space=pltpu.VMEM)] * 5,
        out_specs=pl.BlockSpec(memory_space=pltpu.VMEM),
        scratch_shapes=[
            pltpu.VMEM((N_DEV, 2, Skv, HD), jnp.float32),
            pltpu.VMEM((N_DEV, 2, Skv, HD), jnp.float32),
            pltpu.VMEM((B, Sq, HD), jnp.float32),
            pltpu.VMEM((B, Sq, HD), jnp.float32),
            pltpu.VMEM((B, Hq, Sq), jnp.float32),
            pltpu.VMEM((B, Hq, Sq), jnp.float32),
            pltpu.SemaphoreType.DMA((N_DEV,)),
            pltpu.SemaphoreType.DMA((N_DEV,)),
            pltpu.SemaphoreType.DMA((N_DEV,)),
            pltpu.SemaphoreType.DMA((N_DEV,)),
        ],
        compiler_params=pltpu.CompilerParams(collective_id=0),
    )(x, Wq, K_ext, V_ext, Wo)
